# baseline (speedup 1.0000x reference)
import functools

import jax
import jax.numpy as jnp
from jax import lax
from jax.experimental import pallas as pl
from jax.experimental.pallas import tpu as pltpu
from jax.experimental.pallas import tpu_sc as plsc

B = 16384
D_ID = 32
D_DESC = 16
L = 20
LP = 24
OUT = 32
VD = 1001
ZERO_COL = VD
DT_COLS = 1024

PACK = 128 // D_ID
TROWS = 25088

NW = 32
BPW = B // NW
GCH = 128
NG = BPW // GCH
NCHUNK = BPW // 16


def _sc_body(tab128, ids2, tokt, dtp,
             idrows_out, pooled_out,
             idxhi_v, idsv, id128_v, idrows_v, tok_v, dt_v, pooled_v,
             sem, sem2):
    info = plsc.get_sparse_core_info()
    nc = info.num_cores
    wid = lax.axis_index("s") * nc + lax.axis_index("c")
    base = wid * BPW

    cp_v = pltpu.async_copy(ids2.at[pl.ds(wid * NG, NG)], idsv, sem2)
    cp_v.wait()
    for j in range(NG):
        for k in range(GCH // 16):
            hi = lax.shift_right_logical(idsv[j, pl.ds(k * 16, 16)], 2)
            idxhi_v[j, pl.ds(k * 16, 16)] = hi

    copies = [
        pltpu.async_copy(tab128.at[idxhi_v.at[j]],
                         id128_v.at[pl.ds(j * GCH, GCH)], sem)
        for j in range(NG)
    ]

    cp_dt = pltpu.async_copy(dtp, dt_v, sem2)
    tok_cps = [
        pltpu.async_copy(tokt.at[l, pl.ds(base, BPW)],
                         tok_v.at[pl.ds(l * BPW, BPW)], sem2)
        for l in range(L)
    ]
    cp_dt.wait()
    for cp in tok_cps:
        cp.wait()

    def chunk(c, carry):
        cbase = c * 16
        count = jnp.zeros((16,), jnp.float32)
        accs = [jnp.zeros((16,), jnp.float32) for _ in range(D_DESC)]
        for l in range(L):
            tk = tok_v[pl.ds(l * BPW + cbase, 16)]
            m = tk != 0
            count = count + jnp.where(m, 1.0, 0.0)
            tk = jnp.where(m, tk, jnp.full((16,), ZERO_COL, jnp.int32))
            for d in range(D_DESC):
                accs[d] = accs[d] + plsc.load_gather(
                    dt_v, [tk + d * DT_COLS])
        inv = 1.0 / jnp.maximum(count, 1.0)
        for d in range(D_DESC):
            pooled_v[pl.ds(d * BPW + cbase, 16)] = accs[d] * inv
        return carry

    lax.fori_loop(0, NCHUNK, chunk, 0)

    for cp in copies:
        cp.wait()

    lane = lax.iota(jnp.int32, 16)

    def pick(r, carry):
        rv = jnp.full((16,), r, jnp.int32)
        lo_vec = (plsc.load_gather(idsv, [rv >> 7, rv & (GCH - 1)])
                  & (PACK - 1)) * D_ID
        col = lo_vec + lane
        idrows_v[r, pl.ds(0, 16)] = plsc.load_gather(id128_v, [rv, col])
        idrows_v[r, pl.ds(16, 16)] = plsc.load_gather(id128_v, [rv, col + 16])
        return carry

    lax.fori_loop(0, BPW, pick, 0)

    pltpu.sync_copy(idrows_v, idrows_out.at[pl.ds(base, BPW)])
    out_cps = [
        pltpu.async_copy(pooled_v.at[pl.ds(d * BPW, BPW)],
                         pooled_out.at[d, pl.ds(base, BPW)], sem2)
        for d in range(D_DESC)
    ]
    for cp in out_cps:
        cp.wait()


def _tc_body(idr, prt, w1, w2, bb, out):
    out[...] = (
        jnp.dot(idr[...], w1[...], preferred_element_type=jnp.float32)
        + lax.dot_general(prt[...], w2[...], (((0,), (0,)), ((), ())),
                          preferred_element_type=jnp.float32)
        + bb[...]
    )


_BM = 2048


def kernel(strategy_id, description_tokens, strategy_table, desc_table, W, b):
    ids2 = strategy_id.astype(jnp.int32).reshape(B // GCH, GCH)
    tokt = jnp.pad(description_tokens.astype(jnp.int32).T,
                   ((0, LP - L), (0, 0)))
    dtp = jnp.concatenate(
        [desc_table.T, jnp.zeros((D_DESC, DT_COLS - VD), jnp.float32)],
        axis=1).reshape(D_DESC * DT_COLS)

    tab128 = jnp.pad(
        strategy_table,
        ((0, TROWS * PACK - strategy_table.shape[0]), (0, 0))
    ).reshape(TROWS, 128)

    sc = pl.kernel(
        _sc_body,
        mesh=plsc.VectorSubcoreMesh(core_axis_name="c", subcore_axis_name="s"),
        compiler_params=pltpu.CompilerParams(
            needs_layout_passes=False, use_tc_tiling_on_sc=False),
        out_type=(
            jax.ShapeDtypeStruct((B, D_ID), jnp.float32),
            jax.ShapeDtypeStruct((D_DESC, B), jnp.float32),
        ),
        scratch_types=[
            pltpu.VMEM((NG, GCH), jnp.int32),
            pltpu.VMEM((NG, GCH), jnp.int32),
            pltpu.VMEM((BPW, 128), jnp.float32),
            pltpu.VMEM((BPW, D_ID), jnp.float32),
            pltpu.VMEM((BPW * L,), jnp.int32),
            pltpu.VMEM((D_DESC * DT_COLS,), jnp.float32),
            pltpu.VMEM((D_DESC * BPW,), jnp.float32),
            pltpu.SemaphoreType.DMA,
            pltpu.SemaphoreType.DMA,
        ],
    )
    idrows, pooled_t = sc(tab128, ids2, tokt, dtp)

    w1 = W[:D_ID]
    w2 = W[D_ID:]
    b2 = b.reshape(1, OUT)

    out = pl.pallas_call(
        _tc_body,
        grid=(B // _BM,),
        in_specs=[
            pl.BlockSpec((_BM, D_ID), lambda i: (i, 0)),
            pl.BlockSpec((D_DESC, _BM), lambda i: (0, i)),
            pl.BlockSpec((D_ID, OUT), lambda i: (0, 0)),
            pl.BlockSpec((D_DESC, OUT), lambda i: (0, 0)),
            pl.BlockSpec((1, OUT), lambda i: (0, 0)),
        ],
        out_specs=pl.BlockSpec((_BM, OUT), lambda i: (i, 0)),
        out_shape=jax.ShapeDtypeStruct((B, OUT), jnp.float32),
    )(idrows, pooled_t, w1, w2, b2)
    return out


# speedup vs baseline: 1.3934x; 1.3934x over previous
import functools

import jax
import jax.numpy as jnp
from jax import lax
from jax.experimental import pallas as pl
from jax.experimental.pallas import tpu as pltpu
from jax.experimental.pallas import tpu_sc as plsc

B = 16384
D_ID = 32
D_DESC = 16
L = 20
OUT = 32
VD = 1001
ZERO_COL = VD
DT_COLS = VD + 1

NW = 32
BPW = B // NW
GCH = 128
NG = BPW // GCH
NCHUNK = BPW // 16


def _sc_body(stable, ids2, tokt, dtt,
             idrows_out, pooled_out,
             idx_v, idrows_v, tok_v, dt_v, pooled_v, sem, sem2):
    info = plsc.get_sparse_core_info()
    nc = info.num_cores
    wid = lax.axis_index("s") * nc + lax.axis_index("c")
    base = wid * BPW

    pltpu.sync_copy(ids2.at[pl.ds(wid * NG, NG)], idx_v)
    copies = [
        pltpu.async_copy(stable.at[idx_v.at[j]],
                         idrows_v.at[pl.ds(j * GCH, GCH)], sem)
        for j in range(NG)
    ]

    cp_dt = pltpu.async_copy(dtt, dt_v, sem2)
    tok_cps = [
        pltpu.async_copy(tokt.at[l, pl.ds(base, BPW)],
                         tok_v.at[pl.ds(l * BPW, BPW)], sem2)
        for l in range(L)
    ]
    cp_dt.wait()
    for cp in tok_cps:
        cp.wait()

    def chunk(c, carry):
        cbase = c * 16
        count = jnp.zeros((16,), jnp.float32)
        accs = [jnp.zeros((16,), jnp.float32) for _ in range(D_DESC)]
        for l in range(L):
            tk = tok_v[pl.ds(l * BPW + cbase, 16)]
            m = tk != 0
            count = count + jnp.where(m, 1.0, 0.0)
            tk = jnp.where(m, tk, jnp.full((16,), ZERO_COL, jnp.int32))
            for d in range(D_DESC):
                accs[d] = accs[d] + plsc.load_gather(dt_v, [tk + d * DT_COLS])
        inv = 1.0 / jnp.maximum(count, 1.0)
        for d in range(D_DESC):
            pooled_v[pl.ds(d * BPW + cbase, 16)] = accs[d] * inv
        return carry

    lax.fori_loop(0, NCHUNK, chunk, 0)

    out_cps = [
        pltpu.async_copy(pooled_v.at[pl.ds(d * BPW, BPW)],
                         pooled_out.at[d, pl.ds(base, BPW)], sem2)
        for d in range(D_DESC)
    ]
    for cp in copies:
        cp.wait()
    pltpu.sync_copy(idrows_v, idrows_out.at[pl.ds(base, BPW)])
    for cp in out_cps:
        cp.wait()


def _tc_body(idr, prt, w1, w2, bb, out):
    out[...] = (
        jnp.dot(idr[...], w1[...], preferred_element_type=jnp.float32)
        + lax.dot_general(prt[...], w2[...], (((0,), (0,)), ((), ())),
                          preferred_element_type=jnp.float32)
        + bb[...]
    )


_BM = 2048


def kernel(strategy_id, description_tokens, strategy_table, desc_table, W, b):
    ids2 = strategy_id.astype(jnp.int32).reshape(B // GCH, GCH)
    tokt = description_tokens.astype(jnp.int32).T
    dtt = jnp.concatenate(
        [desc_table.T, jnp.zeros((D_DESC, 1), jnp.float32)],
        axis=1).reshape(D_DESC * DT_COLS)

    sc = pl.kernel(
        _sc_body,
        mesh=plsc.VectorSubcoreMesh(core_axis_name="c", subcore_axis_name="s"),
        compiler_params=pltpu.CompilerParams(
            needs_layout_passes=False, use_tc_tiling_on_sc=False,
            skip_device_barrier=True),
        out_type=(
            jax.ShapeDtypeStruct((B, D_ID), jnp.float32),
            jax.ShapeDtypeStruct((D_DESC, B), jnp.float32),
        ),
        scratch_types=[
            pltpu.VMEM((NG, GCH), jnp.int32),
            pltpu.VMEM((BPW, D_ID), jnp.float32),
            pltpu.VMEM((BPW * L,), jnp.int32),
            pltpu.VMEM((D_DESC * DT_COLS,), jnp.float32),
            pltpu.VMEM((D_DESC * BPW,), jnp.float32),
            pltpu.SemaphoreType.DMA,
            pltpu.SemaphoreType.DMA,
        ],
    )
    idrows, pooled_t = sc(strategy_table, ids2, tokt, dtt)

    w1 = W[:D_ID]
    w2 = W[D_ID:]
    b2 = b.reshape(1, OUT)

    out = pl.pallas_call(
        _tc_body,
        grid=(B // _BM,),
        in_specs=[
            pl.BlockSpec((_BM, D_ID), lambda i: (i, 0)),
            pl.BlockSpec((D_DESC, _BM), lambda i: (0, i)),
            pl.BlockSpec((D_ID, OUT), lambda i: (0, 0)),
            pl.BlockSpec((D_DESC, OUT), lambda i: (0, 0)),
            pl.BlockSpec((1, OUT), lambda i: (0, 0)),
        ],
        out_specs=pl.BlockSpec((_BM, OUT), lambda i: (i, 0)),
        out_shape=jax.ShapeDtypeStruct((B, OUT), jnp.float32),
    )(idrows, pooled_t, w1, w2, b2)
    return out


# speedup vs baseline: 1.3956x; 1.0016x over previous
import functools

import jax
import jax.numpy as jnp
from jax import lax
from jax.experimental import pallas as pl
from jax.experimental.pallas import tpu as pltpu
from jax.experimental.pallas import tpu_sc as plsc

B = 16384
D_ID = 32
D_DESC = 16
L = 20
OUT = 32
VD = 1001
ZERO_COL = VD
DT_COLS = VD + 1

NW = 32
BPW = B // NW
GCH = 128
NG = BPW // GCH
NCHUNK = BPW // 16


def _sc_body(stable, ids2, tokt, dtt,
             idrows_out, pooled_out,
             idx_v, idrows_v, tok_v, dt_v, pooled_v, sem, sem2):
    info = plsc.get_sparse_core_info()
    nc = info.num_cores
    wid = lax.axis_index("s") * nc + lax.axis_index("c")
    base = wid * BPW

    pltpu.sync_copy(ids2.at[pl.ds(wid * NG, NG)], idx_v)
    copies = [
        pltpu.async_copy(stable.at[idx_v.at[j]],
                         idrows_v.at[pl.ds(j * GCH, GCH)], sem)
        for j in range(NG)
    ]

    cp_dt = pltpu.async_copy(dtt, dt_v, sem2)
    tok_cps = [
        pltpu.async_copy(tokt.at[l, pl.ds(base, BPW)],
                         tok_v.at[pl.ds(l * BPW, BPW)], sem2)
        for l in range(L)
    ]
    cp_dt.wait()
    for cp in tok_cps:
        cp.wait()

    def chunk(c, carry):
        cbase = c * 16
        count = jnp.zeros((16,), jnp.float32)
        accs = [jnp.zeros((16,), jnp.float32) for _ in range(D_DESC)]
        for l in range(L):
            tk = tok_v[pl.ds(l * BPW + cbase, 16)]
            m = tk != 0
            count = count + jnp.where(m, 1.0, 0.0)
            tk = jnp.where(m, tk, jnp.full((16,), ZERO_COL, jnp.int32))
            for d in range(D_DESC):
                accs[d] = accs[d] + plsc.load_gather(dt_v, [tk + d * DT_COLS])
        inv = 1.0 / jnp.maximum(count, 1.0)
        for d in range(D_DESC):
            pooled_v[pl.ds(d * BPW + cbase, 16)] = accs[d] * inv
        return carry

    lax.fori_loop(0, NCHUNK, chunk, 0)

    out_cps = [
        pltpu.async_copy(pooled_v.at[pl.ds(d * BPW, BPW)],
                         pooled_out.at[d, pl.ds(base, BPW)], sem2)
        for d in range(D_DESC)
    ]
    for cp in copies:
        cp.wait()
    pltpu.sync_copy(idrows_v, idrows_out.at[pl.ds(base, BPW)])
    for cp in out_cps:
        cp.wait()


def _tc_body(idr, prt, w1, w2, bb, out):
    out[...] = (
        jnp.dot(idr[...], w1[...], preferred_element_type=jnp.float32)
        + lax.dot_general(prt[...], w2[...], (((0,), (0,)), ((), ())),
                          preferred_element_type=jnp.float32)
        + bb[...]
    )


_BM = 2048


def kernel(strategy_id, description_tokens, strategy_table, desc_table, W, b):
    ids2 = strategy_id.astype(jnp.int32).reshape(B // GCH, GCH)
    tokt = description_tokens.astype(jnp.int32).T
    dtt = jnp.concatenate(
        [desc_table.T, jnp.zeros((D_DESC, 1), jnp.float32)],
        axis=1).reshape(D_DESC * DT_COLS)

    sc = pl.kernel(
        _sc_body,
        mesh=plsc.VectorSubcoreMesh(core_axis_name="c", subcore_axis_name="s"),
        compiler_params=pltpu.CompilerParams(
            needs_layout_passes=False, use_tc_tiling_on_sc=False),
        out_type=(
            jax.ShapeDtypeStruct((B, D_ID), jnp.float32),
            jax.ShapeDtypeStruct((D_DESC, B), jnp.float32),
        ),
        scratch_types=[
            pltpu.VMEM((NG, GCH), jnp.int32),
            pltpu.VMEM((BPW, D_ID), jnp.float32),
            pltpu.VMEM((BPW * L,), jnp.int32),
            pltpu.VMEM((D_DESC * DT_COLS,), jnp.float32),
            pltpu.VMEM((D_DESC * BPW,), jnp.float32),
            pltpu.SemaphoreType.DMA,
            pltpu.SemaphoreType.DMA,
        ],
    )
    idrows, pooled_t = sc(strategy_table, ids2, tokt, dtt)

    w1 = W[:D_ID]
    w2 = W[D_ID:]
    b2 = b.reshape(1, OUT)

    out = pl.pallas_call(
        _tc_body,
        grid=(B // _BM,),
        in_specs=[
            pl.BlockSpec((_BM, D_ID), lambda i: (i, 0)),
            pl.BlockSpec((D_DESC, _BM), lambda i: (0, i)),
            pl.BlockSpec((D_ID, OUT), lambda i: (0, 0)),
            pl.BlockSpec((D_DESC, OUT), lambda i: (0, 0)),
            pl.BlockSpec((1, OUT), lambda i: (0, 0)),
        ],
        out_specs=pl.BlockSpec((_BM, OUT), lambda i: (i, 0)),
        out_shape=jax.ShapeDtypeStruct((B, OUT), jnp.float32),
    )(idrows, pooled_t, w1, w2, b2)
    return out
